# Initial kernel scaffold; baseline (speedup 1.0000x reference)
import functools

import jax
import jax.numpy as jnp
from jax import lax
from jax.experimental import pallas as pl
from jax.experimental.pallas import tpu as pltpu
from jax.experimental.pallas import tpu_sc as plsc

_NC = 2
_NS = 16
_NW = _NC * _NS
_CHUNK = 128


def _gather_feature(tbl, idx2, out, idx_v, rows_v, sem, wid, rows_per_w, kr):
    @pl.loop(0, rows_per_w // kr)
    def _(i):
        r0 = wid * rows_per_w + i * kr
        pltpu.sync_copy(idx2.at[pl.ds(r0, kr)], idx_v.at[pl.ds(0, kr)])
        cps = [
            pltpu.async_copy(
                tbl.at[idx_v.at[j]], rows_v.at[pl.ds(j * _CHUNK, _CHUNK)], sem
            )
            for j in range(kr)
        ]
        for c in cps:
            c.wait()
        pltpu.sync_copy(
            rows_v.at[pl.ds(0, kr * _CHUNK)], out.at[pl.ds(r0 * _CHUNK, kr * _CHUNK)]
        )


def _sc_gather_all(tables, idx_lists, out_shapes, plans):
    mesh = plsc.VectorSubcoreMesh(core_axis_name="c", subcore_axis_name="s")
    n = len(tables)
    max64 = 640

    def body(*refs):
        tbls = refs[:n]
        idxs = refs[n : 2 * n]
        outs = refs[2 * n : 3 * n]
        idx_v, rows64_v, rows32_v, sem = refs[3 * n :]
        wid = lax.axis_index("s") * _NC + lax.axis_index("c")
        for f in range(n):
            rows_per_w, kr = plans[f]
            d = out_shapes[f][1]
            rows_v = rows64_v if d == 64 else rows32_v
            if rows_per_w > 0:
                _gather_feature(
                    tbls[f], idxs[f], outs[f], idx_v, rows_v, sem, wid, rows_per_w, kr
                )
            else:
                nrows = idx_lists[f].shape[0]
                tbl_f, idx_f, out_f, rv = tbls[f], idxs[f], outs[f], rows_v

                @pl.when(wid < nrows)
                def _():
                    pltpu.sync_copy(idx_f.at[pl.ds(wid, 1)], idx_v.at[pl.ds(0, 1)])
                    pltpu.async_copy(
                        tbl_f.at[idx_v.at[0]], rv.at[pl.ds(0, _CHUNK)], sem
                    ).wait()
                    pltpu.sync_copy(
                        rv.at[pl.ds(0, _CHUNK)],
                        out_f.at[pl.ds(wid * _CHUNK, _CHUNK)],
                    )

    k = pl.kernel(
        body,
        out_type=[jax.ShapeDtypeStruct(s, jnp.float32) for s in out_shapes],
        mesh=mesh,
        scratch_types=[
            pltpu.VMEM((8, _CHUNK), jnp.int32),
            pltpu.VMEM((max64, 64), jnp.float32),
            pltpu.VMEM((max64, 32), jnp.float32),
            pltpu.SemaphoreType.DMA,
        ],
    )
    return k(*tables, *idx_lists)


def _user_tower_body(
    emb_user, usm, d3, U1id, U1a, U1b, U1c, U1d, Ub1, U2, Ub2, Wc2, Wmg, bmg,
    out, *, B, A
):
    u200 = usm[0:B, :]
    u201 = usm[B : 2 * B, :]
    u202 = usm[2 * B : 3 * B, :]
    for a in range(1, A):
        u202 = u202 + usm[(2 + a) * B : (3 + a) * B, :]
    dot = functools.partial(jnp.dot, preferred_element_type=jnp.float32)
    h = (
        dot(emb_user[...], U1id[...])
        + dot(u200, U1a[...])
        + dot(u201, U1b[...])
        + dot(u202, U1c[...])
        + d3[...] * U1d[...]
        + Ub1[...]
    )
    user_h = dot(jnp.maximum(h, 0.0), U2[...]) + Ub2[...]
    out[...] = dot(dot(user_h, Wc2[...]), Wmg[...]) + bmg[...]


def _main_body(
    e_id, e_210, e_101, e_401, e_100, e_102, e_300, e_301, m1, dd, u3, ee,
    Wm, bm, I1id, I1a, I1b, I1c, I1dd, I1mm, Ib1, I2, Ib2,
    C1a, C1b, C1c, C1d, Cb1, C2, Cb2, Wc1, Wc3, bc, Wmg, out
):
    dot = functools.partial(jnp.dot, preferred_element_type=jnp.float32)
    mm = dot(m1[...], Wm[...]) + bm[...]
    h = (
        dot(e_id[...], I1id[...])
        + dot(e_100[...], I1a[...])
        + dot(e_101[...], I1b[...])
        + dot(e_102[...], I1c[...])
        + dot(dd[...], I1dd[...])
        + dot(mm, I1mm[...])
        + Ib1[...]
    )
    item_h = dot(jnp.maximum(h, 0.0), I2[...]) + Ib2[...]
    hc = (
        dot(e_300[...], C1a[...])
        + dot(e_301[...], C1b[...])
        + dot(e_210[...], C1c[...])
        + dot(e_401[...], C1d[...])
        + Cb1[...]
    )
    ctx_h = dot(jnp.maximum(hc, 0.0), C2[...]) + Cb2[...]
    pre = dot(item_h, Wc1[...]) + dot(ctx_h, Wc3[...]) + bc[...]
    out[...] = dot(pre, Wmg[...]) + dot(ee[...], u3[...])


def kernel(user_id, uf_200, uf_201, uf_202, uf_d3, input_ids, if_100, if_101,
           if_102, if_d1, if_d2, if_m1, cf_300, cf_301, cf_210, cf_401,
           T_item, T101, T100, T102, T_user, T200, T201, T202, T300, T301,
           Wm, bm, I1, Ib1, I2, Ib2, U1, Ub1, U2, Ub2, C1, Cb1, C2, Cb2,
           Wc, bc, Wmg, bmg):
    B, L = input_ids.shape
    A = uf_202.shape[1]
    BL = B * L
    D_ID = T_item.shape[1]
    D_F = T101.shape[1]
    H = Wmg.shape[0]
    V_SM1 = T200.shape[0]

    idx_item = jnp.concatenate(
        [input_ids.reshape(-1), cf_210.reshape(-1)]
    ).reshape(-1, _CHUNK)
    idx_101 = jnp.concatenate(
        [if_101.reshape(-1), cf_401.reshape(-1)]
    ).reshape(-1, _CHUNK)
    idx_100 = if_100.reshape(-1, _CHUNK)
    idx_102 = if_102.reshape(-1, _CHUNK)
    idx_300 = cf_300.reshape(-1, _CHUNK)
    idx_301 = cf_301.reshape(-1, _CHUNK)
    T_usm = jnp.concatenate([T200, T201, T202], axis=0)
    idx_usm = jnp.concatenate(
        [uf_200, uf_201 + V_SM1, (uf_202.T.reshape(-1) + 2 * V_SM1)]
    ).reshape(-1, _CHUNK)
    idx_user = user_id.reshape(-1, _CHUNK)

    n_item_rows = idx_item.shape[0]
    n_f_rows = idx_100.shape[0]
    n_usm_rows = idx_usm.shape[0]

    emb_item, emb_101, emb_100, emb_102, emb_300, emb_301, emb_usm, emb_user = (
        _sc_gather_all(
            tables=[T_item, T101, T100, T102, T300, T301, T_usm, T_user],
            idx_lists=[idx_item, idx_101, idx_100, idx_102, idx_300, idx_301,
                       idx_usm, idx_user],
            out_shapes=[
                (n_item_rows * _CHUNK, D_ID), (n_item_rows * _CHUNK, D_F),
                (n_f_rows * _CHUNK, D_F), (n_f_rows * _CHUNK, D_F),
                (n_f_rows * _CHUNK, D_F), (n_f_rows * _CHUNK, D_F),
                (n_usm_rows * _CHUNK, D_F), (B, D_ID),
            ],
            plans=[
                (n_item_rows // _NW, 5), (n_item_rows // _NW, 5),
                (n_f_rows // _NW, 5), (n_f_rows // _NW, 5),
                (n_f_rows // _NW, 5), (n_f_rows // _NW, 5),
                (n_usm_rows // _NW, 3), (0, 0),
            ],
        )
    )

    f32 = jnp.float32
    I1id, I1a, I1b, I1c = I1[:D_ID], I1[D_ID:D_ID + D_F], \
        I1[D_ID + D_F:D_ID + 2 * D_F], I1[D_ID + 2 * D_F:D_ID + 3 * D_F]
    I1dd = I1[D_ID + 3 * D_F:D_ID + 3 * D_F + 2]
    I1mm = I1[D_ID + 3 * D_F + 2:]
    U1id, U1a, U1b, U1c = U1[:D_ID], U1[D_ID:D_ID + D_F], \
        U1[D_ID + D_F:D_ID + 2 * D_F], U1[D_ID + 2 * D_F:D_ID + 3 * D_F]
    U1d = U1[D_ID + 3 * D_F:]
    C1a, C1b = C1[:D_F], C1[D_F:2 * D_F]
    C1c, C1d = C1[2 * D_F:2 * D_F + D_ID], C1[2 * D_F + D_ID:]
    Wc1, Wc2, Wc3 = Wc[:H], Wc[H:2 * H], Wc[2 * H:]
    Ib1_2, Ib2_2 = Ib1.reshape(1, -1), Ib2.reshape(1, -1)
    Ub1_2, Ub2_2 = Ub1.reshape(1, -1), Ub2.reshape(1, -1)
    Cb1_2, Cb2_2 = Cb1.reshape(1, -1), Cb2.reshape(1, -1)
    bm_2, bc_2, bmg_2 = bm.reshape(1, -1), bc.reshape(1, -1), bmg.reshape(1, -1)
    d3_2 = uf_d3.reshape(-1, 1)
    dd = jnp.stack([if_d1.reshape(-1), if_d2.reshape(-1)], axis=-1)
    m1 = if_m1.reshape(BL, -1)

    u3 = pl.pallas_call(
        functools.partial(_user_tower_body, B=B, A=A),
        out_shape=jax.ShapeDtypeStruct((B, H), f32),
    )(emb_user, emb_usm, d3_2, U1id, U1a, U1b, U1c, U1d, Ub1_2, U2, Ub2_2,
      Wc2, Wmg, bmg_2)

    BBLK = 8
    M = BBLK * L
    NB = B // BBLK
    ee = (jnp.arange(M)[:, None] // L == jnp.arange(BBLK)[None, :]).astype(f32)

    def rowblk(d):
        return pl.BlockSpec((M, d), lambda m: (m, 0))

    def rowblk_off(d, off_blocks):
        return pl.BlockSpec((M, d), lambda m: (m + off_blocks, 0))

    def full(a):
        return pl.BlockSpec(a.shape, lambda m: tuple(0 for _ in a.shape))

    weights = [Wm, bm_2, I1id, I1a, I1b, I1c, I1dd, I1mm, Ib1_2, I2, Ib2_2,
               C1a, C1b, C1c, C1d, Cb1_2, C2, Cb2_2, Wc1, Wc3, bc_2, Wmg]
    out = pl.pallas_call(
        _main_body,
        grid=(NB,),
        in_specs=[
            rowblk(D_ID), rowblk_off(D_ID, NB),
            rowblk(D_F), rowblk_off(D_F, NB),
            rowblk(D_F), rowblk(D_F),
            rowblk(D_F), rowblk(D_F),
            rowblk(if_m1.shape[2]),
            rowblk(2),
            pl.BlockSpec((BBLK, H), lambda m: (m, 0)),
            full(ee),
        ] + [full(w) for w in weights],
        out_specs=rowblk(H),
        out_shape=jax.ShapeDtypeStruct((BL, H), f32),
    )(emb_item, emb_item, emb_101, emb_101, emb_100, emb_102, emb_300, emb_301,
      m1, dd, u3, ee, *weights)

    return out.reshape(B, L, H)


# speedup vs baseline: 2.7658x; 2.7658x over previous
import functools

import jax
import jax.numpy as jnp
from jax import lax
from jax.experimental import pallas as pl
from jax.experimental.pallas import tpu as pltpu
from jax.experimental.pallas import tpu_sc as plsc

_NC = 2
_NS = 16
_NW = _NC * _NS
_CHUNK = 128


_KR = 8


def _gather_feature(tbl, idx2, out, idx_v, rows_v, sem, wid, total_steps):
    n_loop = -(-total_steps // _NW)

    @pl.loop(0, n_loop)
    def _(i):
        t = wid + i * _NW

        @pl.when(t < total_steps)
        def _():
            r0 = pl.multiple_of(t * _KR, _KR)
            pltpu.sync_copy(idx2.at[pl.ds(r0, _KR)], idx_v)
            cps = [
                pltpu.async_copy(
                    tbl.at[idx_v.at[j]], rows_v.at[pl.ds(j * _CHUNK, _CHUNK)], sem
                )
                for j in range(_KR)
            ]
            for c in cps:
                c.wait()
            pltpu.sync_copy(rows_v, out.at[pl.ds(r0 * _CHUNK, _KR * _CHUNK)])


def _sc_gather_all(tables, idx_lists, out_shapes):
    mesh = plsc.VectorSubcoreMesh(core_axis_name="c", subcore_axis_name="s")
    n = len(tables)

    def body(*refs):
        tbls = refs[:n]
        idxs = refs[n : 2 * n]
        outs = refs[2 * n : 3 * n]
        idx_v, rows64_v, rows32_v, sem = refs[3 * n :]
        wid = lax.axis_index("s") * _NC + lax.axis_index("c")
        for f in range(n):
            d = out_shapes[f][1]
            rows_v = rows64_v if d == 64 else rows32_v
            total_steps = idx_lists[f].shape[0] // _KR
            _gather_feature(
                tbls[f], idxs[f], outs[f], idx_v, rows_v, sem, wid, total_steps
            )

    k = pl.kernel(
        body,
        out_type=[jax.ShapeDtypeStruct(s, jnp.float32) for s in out_shapes],
        mesh=mesh,
        compiler_params=pltpu.CompilerParams(use_tc_tiling_on_sc=False),
        scratch_types=[
            pltpu.VMEM((_KR, _CHUNK), jnp.int32),
            pltpu.VMEM((_KR * _CHUNK, 64), jnp.float32),
            pltpu.VMEM((_KR * _CHUNK, 32), jnp.float32),
            pltpu.SemaphoreType.DMA,
        ],
    )
    return k(*tables, *idx_lists)


def _user_tower_body(
    emb_user, usm, d3, U1id, U1a, U1b, U1c, U1d, Ub1, U2, Ub2, Wc2, Wmg, bmg,
    out, *, B, A
):
    u200 = usm[0:B, :]
    u201 = usm[B : 2 * B, :]
    u202 = usm[2 * B : 3 * B, :]
    for a in range(1, A):
        u202 = u202 + usm[(2 + a) * B : (3 + a) * B, :]
    dot = functools.partial(jnp.dot, preferred_element_type=jnp.float32)
    h = (
        dot(emb_user[...], U1id[...])
        + dot(u200, U1a[...])
        + dot(u201, U1b[...])
        + dot(u202, U1c[...])
        + d3[...] * U1d[...]
        + Ub1[...]
    )
    user_h = dot(jnp.maximum(h, 0.0), U2[...]) + Ub2[...]
    out[...] = dot(dot(user_h, Wc2[...]), Wmg[...]) + bmg[...]


def _main_body(
    e_id, e_210, e_101, e_401, e_100, e_102, e_300, e_301, m1, dd, u3, ee,
    Wm, bm, I1id, I1a, I1b, I1c, I1dd, I1mm, Ib1, I2, Ib2,
    C1a, C1b, C1c, C1d, Cb1, C2, Cb2, Wc1, Wc3, bc, Wmg, out
):
    dot = functools.partial(jnp.dot, preferred_element_type=jnp.float32)
    mm = dot(m1[...], Wm[...]) + bm[...]
    h = (
        dot(e_id[...], I1id[...])
        + dot(e_100[...], I1a[...])
        + dot(e_101[...], I1b[...])
        + dot(e_102[...], I1c[...])
        + dot(dd[...], I1dd[...])
        + dot(mm, I1mm[...])
        + Ib1[...]
    )
    item_h = dot(jnp.maximum(h, 0.0), I2[...]) + Ib2[...]
    hc = (
        dot(e_300[...], C1a[...])
        + dot(e_301[...], C1b[...])
        + dot(e_210[...], C1c[...])
        + dot(e_401[...], C1d[...])
        + Cb1[...]
    )
    ctx_h = dot(jnp.maximum(hc, 0.0), C2[...]) + Cb2[...]
    pre = dot(item_h, Wc1[...]) + dot(ctx_h, Wc3[...]) + bc[...]
    out[...] = dot(pre, Wmg[...]) + dot(ee[...], u3[...])


def kernel(user_id, uf_200, uf_201, uf_202, uf_d3, input_ids, if_100, if_101,
           if_102, if_d1, if_d2, if_m1, cf_300, cf_301, cf_210, cf_401,
           T_item, T101, T100, T102, T_user, T200, T201, T202, T300, T301,
           Wm, bm, I1, Ib1, I2, Ib2, U1, Ub1, U2, Ub2, C1, Cb1, C2, Cb2,
           Wc, bc, Wmg, bmg):
    B, L = input_ids.shape
    A = uf_202.shape[1]
    BL = B * L
    D_ID = T_item.shape[1]
    D_F = T101.shape[1]
    H = Wmg.shape[0]
    V_SM1 = T200.shape[0]

    idx_item = jnp.concatenate(
        [input_ids.reshape(-1), cf_210.reshape(-1)]
    ).reshape(-1, _CHUNK)
    idx_101 = jnp.concatenate(
        [if_101.reshape(-1), cf_401.reshape(-1)]
    ).reshape(-1, _CHUNK)
    idx_100 = if_100.reshape(-1, _CHUNK)
    idx_102 = if_102.reshape(-1, _CHUNK)
    idx_300 = cf_300.reshape(-1, _CHUNK)
    idx_301 = cf_301.reshape(-1, _CHUNK)
    T_usm = jnp.concatenate([T200, T201, T202], axis=0)
    idx_usm = jnp.concatenate(
        [uf_200, uf_201 + V_SM1, (uf_202.T.reshape(-1) + 2 * V_SM1)]
    ).reshape(-1, _CHUNK)
    idx_user = user_id.reshape(-1, _CHUNK)

    n_item_rows = idx_item.shape[0]
    n_f_rows = idx_100.shape[0]
    n_usm_rows = idx_usm.shape[0]

    emb_item, emb_101, emb_100, emb_102, emb_300, emb_301, emb_usm, emb_user = (
        _sc_gather_all(
            tables=[T_item, T101, T100, T102, T300, T301, T_usm, T_user],
            idx_lists=[idx_item, idx_101, idx_100, idx_102, idx_300, idx_301,
                       idx_usm, idx_user],
            out_shapes=[
                (n_item_rows * _CHUNK, D_ID), (n_item_rows * _CHUNK, D_F),
                (n_f_rows * _CHUNK, D_F), (n_f_rows * _CHUNK, D_F),
                (n_f_rows * _CHUNK, D_F), (n_f_rows * _CHUNK, D_F),
                (n_usm_rows * _CHUNK, D_F), (B, D_ID),
            ],
        )
    )

    f32 = jnp.float32
    I1id, I1a, I1b, I1c = I1[:D_ID], I1[D_ID:D_ID + D_F], \
        I1[D_ID + D_F:D_ID + 2 * D_F], I1[D_ID + 2 * D_F:D_ID + 3 * D_F]
    I1dd = I1[D_ID + 3 * D_F:D_ID + 3 * D_F + 2]
    I1mm = I1[D_ID + 3 * D_F + 2:]
    U1id, U1a, U1b, U1c = U1[:D_ID], U1[D_ID:D_ID + D_F], \
        U1[D_ID + D_F:D_ID + 2 * D_F], U1[D_ID + 2 * D_F:D_ID + 3 * D_F]
    U1d = U1[D_ID + 3 * D_F:]
    C1a, C1b = C1[:D_F], C1[D_F:2 * D_F]
    C1c, C1d = C1[2 * D_F:2 * D_F + D_ID], C1[2 * D_F + D_ID:]
    Wc1, Wc2, Wc3 = Wc[:H], Wc[H:2 * H], Wc[2 * H:]
    Ib1_2, Ib2_2 = Ib1.reshape(1, -1), Ib2.reshape(1, -1)
    Ub1_2, Ub2_2 = Ub1.reshape(1, -1), Ub2.reshape(1, -1)
    Cb1_2, Cb2_2 = Cb1.reshape(1, -1), Cb2.reshape(1, -1)
    bm_2, bc_2, bmg_2 = bm.reshape(1, -1), bc.reshape(1, -1), bmg.reshape(1, -1)
    d3_2 = uf_d3.reshape(-1, 1)
    dd = jnp.stack([if_d1.reshape(-1), if_d2.reshape(-1)], axis=-1)
    m1 = if_m1.reshape(BL, -1)

    u3 = pl.pallas_call(
        functools.partial(_user_tower_body, B=B, A=A),
        out_shape=jax.ShapeDtypeStruct((B, H), f32),
    )(emb_user, emb_usm, d3_2, U1id, U1a, U1b, U1c, U1d, Ub1_2, U2, Ub2_2,
      Wc2, Wmg, bmg_2)

    BBLK = 8
    M = BBLK * L
    NB = B // BBLK
    ee = (jnp.arange(M)[:, None] // L == jnp.arange(BBLK)[None, :]).astype(f32)

    def rowblk(d):
        return pl.BlockSpec((M, d), lambda m: (m, 0))

    def rowblk_off(d, off_blocks):
        return pl.BlockSpec((M, d), lambda m: (m + off_blocks, 0))

    def full(a):
        return pl.BlockSpec(a.shape, lambda m: tuple(0 for _ in a.shape))

    weights = [Wm, bm_2, I1id, I1a, I1b, I1c, I1dd, I1mm, Ib1_2, I2, Ib2_2,
               C1a, C1b, C1c, C1d, Cb1_2, C2, Cb2_2, Wc1, Wc3, bc_2, Wmg]
    out = pl.pallas_call(
        _main_body,
        grid=(NB,),
        in_specs=[
            rowblk(D_ID), rowblk_off(D_ID, NB),
            rowblk(D_F), rowblk_off(D_F, NB),
            rowblk(D_F), rowblk(D_F),
            rowblk(D_F), rowblk(D_F),
            rowblk(if_m1.shape[2]),
            rowblk(2),
            pl.BlockSpec((BBLK, H), lambda m: (m, 0)),
            full(ee),
        ] + [full(w) for w in weights],
        out_specs=rowblk(H),
        out_shape=jax.ShapeDtypeStruct((BL, H), f32),
    )(emb_item, emb_item, emb_101, emb_101, emb_100, emb_102, emb_300, emb_301,
      m1, dd, u3, ee, *weights)

    return out.reshape(B, L, H)


# speedup vs baseline: 3.5121x; 1.2699x over previous
import functools

import jax
import jax.numpy as jnp
from jax import lax
from jax.experimental import pallas as pl
from jax.experimental.pallas import tpu as pltpu
from jax.experimental.pallas import tpu_sc as plsc

_NC = 2
_NS = 16
_NW = _NC * _NS
_CHUNK = 128
_KR = 8


def _gather_feature(tbl, idx2, row_off, n_rows, out, col0, d, idx_v, rows_v,
                    sem, wid):
    total_steps = n_rows // _KR
    n_loop = -(-total_steps // _NW)

    @pl.loop(0, n_loop)
    def _(i):
        t = wid + i * _NW

        @pl.when(t < total_steps)
        def _():
            r0 = pl.multiple_of(t * _KR, _KR)
            pltpu.sync_copy(idx2.at[pl.ds(row_off + r0, _KR)], idx_v)
            cps = [
                pltpu.async_copy(
                    tbl.at[idx_v.at[j]], rows_v.at[pl.ds(j * _CHUNK, _CHUNK)], sem
                )
                for j in range(_KR)
            ]
            for c in cps:
                c.wait()
            pltpu.sync_copy(
                rows_v,
                out.at[pl.ds(r0 * _CHUNK, _KR * _CHUNK), pl.ds(col0, d)],
            )


def _sc_gather_all(tables, idx_lists, out_shapes, features):
    mesh = plsc.VectorSubcoreMesh(core_axis_name="c", subcore_axis_name="s")
    nt, ni = len(tables), len(idx_lists)

    def body(*refs):
        tbls = refs[:nt]
        idxs = refs[nt : nt + ni]
        outs = refs[nt + ni : nt + ni + len(out_shapes)]
        idx_v, rows64_v, rows32_v, sem = refs[nt + ni + len(out_shapes) :]
        wid = lax.axis_index("s") * _NC + lax.axis_index("c")
        for (tp, ip, roff, nrows, op, col0, d) in features:
            rows_v = rows64_v if d == 64 else rows32_v
            _gather_feature(
                tbls[tp], idxs[ip], roff, nrows, outs[op], col0, d, idx_v,
                rows_v, sem, wid
            )

    k = pl.kernel(
        body,
        out_type=[jax.ShapeDtypeStruct(s, jnp.float32) for s in out_shapes],
        mesh=mesh,
        compiler_params=pltpu.CompilerParams(use_tc_tiling_on_sc=False),
        scratch_types=[
            pltpu.VMEM((_KR, _CHUNK), jnp.int32),
            pltpu.VMEM((_KR * _CHUNK, 64), jnp.float32),
            pltpu.VMEM((_KR * _CHUNK, 32), jnp.float32),
            pltpu.SemaphoreType.DMA,
        ],
    )
    return k(*tables, *idx_lists)


def _user_tower_body(
    emb_user, usm, d3, U1id, U1a, U1b, U1c, U1d, Ub1, U2, Ub2, Wc2, Wmg, bmg,
    out, *, B, A
):
    u200 = usm[0:B, :]
    u201 = usm[B : 2 * B, :]
    u202 = usm[2 * B : 3 * B, :]
    for a in range(1, A):
        u202 = u202 + usm[(2 + a) * B : (3 + a) * B, :]
    dot = functools.partial(jnp.dot, preferred_element_type=jnp.float32)
    h = (
        dot(emb_user[...], U1id[...])
        + dot(u200, U1a[...])
        + dot(u201, U1b[...])
        + dot(u202, U1c[...])
        + d3[...] * U1d[...]
        + Ub1[...]
    )
    user_h = dot(jnp.maximum(h, 0.0), U2[...]) + Ub2[...]
    out[...] = dot(dot(user_h, Wc2[...]), Wmg[...]) + bmg[...]


def _main_body(
    item_cat, ctx_cat, m1, dd, u3, ee,
    I1cat, I1dd, Fmm, Ibe, I2, Ib2, C1, Cb1, C2, Cb2, Wc1, Wc3, bc, Wmg, out,
    *, bblk, l, h_dim
):
    dot = functools.partial(jnp.dot, preferred_element_type=jnp.float32)
    m = bblk * l
    m1f = m1[...].reshape(m, m1.shape[-1])
    h = (
        dot(item_cat[...], I1cat[...])
        + dot(dd[...], I1dd[...])
        + dot(m1f, Fmm[...])
        + Ibe[...]
    )
    item_h = dot(jnp.maximum(h, 0.0), I2[...]) + Ib2[...]
    hc = dot(ctx_cat[...], C1[...]) + Cb1[...]
    ctx_h = dot(jnp.maximum(hc, 0.0), C2[...]) + Cb2[...]
    pre = dot(item_h, Wc1[...]) + dot(ctx_h, Wc3[...]) + bc[...]
    o = dot(pre, Wmg[...]) + dot(ee[...], u3[...])
    out[...] = o.reshape(bblk, l, h_dim)


def kernel(user_id, uf_200, uf_201, uf_202, uf_d3, input_ids, if_100, if_101,
           if_102, if_d1, if_d2, if_m1, cf_300, cf_301, cf_210, cf_401,
           T_item, T101, T100, T102, T_user, T200, T201, T202, T300, T301,
           Wm, bm, I1, Ib1, I2, Ib2, U1, Ub1, U2, Ub2, C1, Cb1, C2, Cb2,
           Wc, bc, Wmg, bmg):
    B, L = input_ids.shape
    A = uf_202.shape[1]
    BL = B * L
    D_ID = T_item.shape[1]
    D_F = T101.shape[1]
    H = Wmg.shape[0]
    V_SM1 = T200.shape[0]
    CAT = D_ID + 3 * D_F

    T_usm = jnp.concatenate([T200, T201, T202], axis=0)
    idx_usm = jnp.concatenate(
        [uf_200, uf_201 + V_SM1, (uf_202.T.reshape(-1) + 2 * V_SM1)]
    ).reshape(-1, _CHUNK)
    user_id2 = user_id.reshape(-1, _CHUNK)
    idx_big = jnp.concatenate(
        [input_ids, if_100, if_101, if_102, cf_300, cf_301, cf_210, cf_401],
        axis=0,
    ).reshape(-1, _CHUNK)
    FR = BL // _CHUNK

    out_item, out_ctx, emb_usm, emb_user = _sc_gather_all(
        tables=[T_item, T100, T101, T102, T300, T301, T_usm, T_user],
        idx_lists=[idx_big, idx_usm, user_id2],
        out_shapes=[(BL, CAT), (BL, CAT), (idx_usm.size, D_F), (B, D_ID)],
        features=[
            (0, 0, 0 * FR, FR, 0, 0, D_ID),
            (1, 0, 1 * FR, FR, 0, D_ID, D_F),
            (2, 0, 2 * FR, FR, 0, D_ID + D_F, D_F),
            (3, 0, 3 * FR, FR, 0, D_ID + 2 * D_F, D_F),
            (4, 0, 4 * FR, FR, 1, 0, D_F),
            (5, 0, 5 * FR, FR, 1, D_F, D_F),
            (0, 0, 6 * FR, FR, 1, 2 * D_F, D_ID),
            (2, 0, 7 * FR, FR, 1, 2 * D_F + D_ID, D_F),
            (6, 1, 0, idx_usm.shape[0], 2, 0, D_F),
            (7, 2, 0, user_id2.shape[0], 3, 0, D_ID),
        ],
    )

    f32 = jnp.float32
    I1cat = I1[:CAT]
    I1dd = I1[CAT:CAT + 2]
    I1mm = I1[CAT + 2:]
    Fmm = Wm @ I1mm
    Ibe = (Ib1 + bm @ I1mm).reshape(1, -1)
    U1id, U1a, U1b, U1c = U1[:D_ID], U1[D_ID:D_ID + D_F], \
        U1[D_ID + D_F:D_ID + 2 * D_F], U1[D_ID + 2 * D_F:D_ID + 3 * D_F]
    U1d = U1[D_ID + 3 * D_F:]
    Wc1, Wc2, Wc3 = Wc[:H], Wc[H:2 * H], Wc[2 * H:]
    Ub1_2, Ub2_2 = Ub1.reshape(1, -1), Ub2.reshape(1, -1)
    Ib2_2 = Ib2.reshape(1, -1)
    Cb1_2, Cb2_2 = Cb1.reshape(1, -1), Cb2.reshape(1, -1)
    bc_2, bmg_2 = bc.reshape(1, -1), bmg.reshape(1, -1)
    d3_2 = uf_d3.reshape(-1, 1)
    dd = jnp.stack([if_d1.reshape(-1), if_d2.reshape(-1)], axis=-1)

    u3 = pl.pallas_call(
        functools.partial(_user_tower_body, B=B, A=A),
        out_shape=jax.ShapeDtypeStruct((B, H), f32),
    )(emb_user, emb_usm, d3_2, U1id, U1a, U1b, U1c, U1d, Ub1_2, U2, Ub2_2,
      Wc2, Wmg, bmg_2)

    BBLK = 8
    M = BBLK * L
    NB = B // BBLK
    MM = if_m1.shape[2]
    ee = (jnp.arange(M)[:, None] // L == jnp.arange(BBLK)[None, :]).astype(f32)

    def rowblk(d):
        return pl.BlockSpec((M, d), lambda m: (m, 0))

    def full(a):
        return pl.BlockSpec(a.shape, lambda m: tuple(0 for _ in a.shape))

    weights = [I1cat, I1dd, Fmm, Ibe, I2, Ib2_2, C1, Cb1_2, C2, Cb2_2,
               Wc1, Wc3, bc_2, Wmg]
    out = pl.pallas_call(
        functools.partial(_main_body, bblk=BBLK, l=L, h_dim=H),
        grid=(NB,),
        in_specs=[
            rowblk(CAT), rowblk(CAT),
            pl.BlockSpec((BBLK, L, MM), lambda m: (m, 0, 0)),
            rowblk(2),
            pl.BlockSpec((BBLK, H), lambda m: (m, 0)),
            full(ee),
        ] + [full(w) for w in weights],
        out_specs=pl.BlockSpec((BBLK, L, H), lambda m: (m, 0, 0)),
        out_shape=jax.ShapeDtypeStruct((B, L, H), f32),
    )(out_item, out_ctx, if_m1, dd, u3, ee, *weights)

    return out
